# baseline (speedup 1.0000x reference)
import functools

import jax
import jax.numpy as jnp
from jax import lax
from jax.experimental import pallas as pl
from jax.experimental.pallas import tpu as pltpu
from jax.experimental.pallas import tpu_sc as plsc

N = 10000
E = 320000
D = 128
L = 16
GROUP = 128
NC = 2
NS = 16
NW = NC * NS
NGT = 80
G_PAD = NW * NGT
E_PAD = G_PAD * GROUP
PCHUNK = 80
NPC = N // PCHUNK
PC_TILE = NPC // NS
PC_REM = NPC - PC_TILE * NS
MM_BLK = 2000


def _sc_layer(x, src, dst, wgt):
    mesh = plsc.VectorSubcoreMesh(core_axis_name="c", subcore_axis_name="s")

    @functools.partial(
        pl.kernel,
        out_type=jax.ShapeDtypeStruct((NC, N, D), jnp.float32),
        mesh=mesh,
        scratch_types=[
            pltpu.VMEM_SHARED((N, D), jnp.float32),
            pltpu.VMEM((GROUP,), jnp.int32),
            pltpu.VMEM((GROUP,), jnp.int32),
            pltpu.VMEM((GROUP,), jnp.int32),
            pltpu.VMEM((GROUP,), jnp.float32),
            pltpu.VMEM((GROUP, D), jnp.float32),
            pltpu.VMEM((GROUP, D), jnp.float32),
            pltpu.VMEM((PCHUNK, D), jnp.float32),
            pltpu.SemaphoreType.DMA,
            pltpu.SemaphoreType.DMA,
        ],
    )
    def sc_kernel(x_hbm, src_hbm, dst_hbm, w_hbm, out_hbm,
                  acc, sv0, sv1, dv, wv, rows0, rows1, stage, sem0, sem1):
        c = lax.axis_index("c")
        s = lax.axis_index("s")
        wid = c * NS + s
        g0 = wid * NGT

        def _zero(r, carry):
            for j in range(D // L):
                stage[r, pl.ds(j * L, L)] = jnp.zeros((L,), jnp.float32)
            return carry
        lax.fori_loop(0, PCHUNK, _zero, 0)
        pc0 = s * PC_TILE + jnp.minimum(s, PC_REM)
        pcnt = PC_TILE + jnp.where(s < PC_REM, 1, 0)

        def _zacc(k, carry):
            pltpu.sync_copy(stage, acc.at[pl.ds((pc0 + k) * PCHUNK, PCHUNK)])
            return carry
        lax.fori_loop(0, pcnt, _zacc, 0)
        plsc.subcore_barrier()

        slots = ((sv0, rows0, sem0), (sv1, rows1, sem1))

        for b, (sv, rows_b, sem) in enumerate(slots):
            pltpu.sync_copy(src_hbm.at[g0 + b], sv)
            pltpu.async_copy(x_hbm.at[sv], rows_b, sem)

        @pl.loop(0, NGT, step=2)
        def _groups(k):
            for b, (sv, rows_b, sem) in enumerate(slots):
                g = g0 + k + b
                pltpu.make_async_copy(
                    x_hbm.at[pl.ds(0, GROUP)], rows_b, sem).wait()
                pltpu.sync_copy(w_hbm.at[g], wv)
                pltpu.sync_copy(dst_hbm.at[g], dv)

                def scale16(eb, carry):
                    wv16 = wv[pl.ds(eb * L, L)]
                    for lane in range(L):
                        wvb = jnp.full((L,), wv16[lane], jnp.float32)
                        e = eb * L + lane
                        for j in range(D // L):
                            rows_b[e, pl.ds(j * L, L)] = (
                                rows_b[e, pl.ds(j * L, L)] * wvb)
                    return carry
                lax.fori_loop(0, GROUP // L, scale16, 0)

                pltpu.sync_copy(rows_b, acc.at[dv], add=True)

                @pl.when(k + b + 2 < NGT)
                def _refill():
                    pltpu.sync_copy(src_hbm.at[g + 2], sv)
                    pltpu.async_copy(x_hbm.at[sv], rows_b, sem)

        plsc.subcore_barrier()

        def _pub(k, carry):
            r0 = (pc0 + k) * PCHUNK
            pltpu.sync_copy(acc.at[pl.ds(r0, PCHUNK)], stage)
            pltpu.sync_copy(stage, out_hbm.at[c, pl.ds(r0, PCHUNK)])
            return carry
        lax.fori_loop(0, pcnt, _pub, 0)

    return sc_kernel(x, src, dst, wgt)


def _mm_fused(p, b, W):
    def body(p_ref, b_ref, w_ref, o_ref):
        hs = p_ref[0] + p_ref[1]
        o_ref[...] = jnp.dot(hs, w_ref[...],
                             preferred_element_type=jnp.float32) + b_ref[...]
    return pl.pallas_call(
        body,
        grid=(N // MM_BLK,),
        in_specs=[pl.BlockSpec((NC, MM_BLK, D), lambda i: (0, i, 0)),
                  pl.BlockSpec((1, D), lambda i: (0, 0)),
                  pl.BlockSpec((D, D), lambda i: (0, 0))],
        out_specs=pl.BlockSpec((MM_BLK, D), lambda i: (i, 0)),
        out_shape=jax.ShapeDtypeStruct((N, D), jnp.float32),
    )(p, b, W)


def kernel(x, edge_index, edge_weight, W1, b1, W2, b2):
    pad = E_PAD - E
    src = jnp.concatenate(
        [edge_index[0], jnp.zeros((pad,), jnp.int32)]).reshape(G_PAD, GROUP)
    dst = jnp.concatenate(
        [edge_index[1], jnp.zeros((pad,), jnp.int32)]).reshape(G_PAD, GROUP)
    wgt = jnp.concatenate(
        [edge_weight, jnp.zeros((pad,), jnp.float32)]).reshape(G_PAD, GROUP)
    b1r = b1.reshape(1, D)
    b2r = b2.reshape(1, D)

    p1 = _sc_layer(x, src, dst, wgt)
    h1 = _mm_fused(p1, b1r, W1)
    p2 = _sc_layer(h1, src, dst, wgt)
    return _mm_fused(p2, b2r, W2)
